# Initial kernel scaffold; baseline (speedup 1.0000x reference)
import jax
import jax.numpy as jnp
from jax.experimental import pallas as pl


def kernel(phi, flat_indices, cu_seqlens):
    raise NotImplementedError("write your pallas kernel here")


# speedup vs baseline: 5.0847x; 5.0847x over previous
import functools
import jax
import jax.numpy as jnp
from jax import lax
from jax.experimental import pallas as pl
from jax.experimental.pallas import tpu as pltpu
from jax.experimental.pallas import tpu_sc as plsc

_NUM_ATOMS = 100000
_TOTAL = 32768
_B = 16
_NC = 2
_NS = 16
_L = 16
_NW = _NC * _NS
_TOK_W = _TOTAL // _NW
_GCH = 128
_NG = _TOK_W // _GCH
_CHW = 3136
_PHI_PAD = _NW * _CHW
_PB = _B * _L

_NEG_INF = float("-inf")
_POS_INF = float("inf")


def _sc_body(phi_hbm, idx_hbm, culo_hbm, cuhi_hbm, outmax_hbm, outmin_hbm,
             culo_v, cuhi_v, idx_v, vals_v, minb_v, vec_v, pm_v, shmax,
             shmin, cmb_v, cmbmin_v, sem):
    cid = lax.axis_index("c")
    sid = lax.axis_index("s")
    wid = sid * _NC + cid
    base = wid * _TOK_W

    pltpu.sync_copy(culo_hbm, culo_v)
    pltpu.sync_copy(cuhi_hbm, cuhi_v)
    pltpu.sync_copy(idx_hbm.at[pl.ds(base, _TOK_W)], idx_v)

    cps = [
        pltpu.async_copy(
            phi_hbm.at[idx_v.at[pl.ds(j * _GCH, _GCH)]],
            vals_v.at[pl.ds(j * _GCH, _GCH)],
            sem,
        )
        for j in range(_NG)
    ]
    for cp in cps:
        cp.wait()

    lane = lax.broadcasted_iota(jnp.int32, (_L,), 0)
    lov = culo_v[...]
    hiv = cuhi_v[...]

    for b in range(_B):
        lo = lov[b]
        hi = hiv[b]
        s0 = jnp.maximum(lo, base)
        e0 = jnp.minimum(hi, base + _TOK_W)
        j0 = (s0 - base) // _L
        j1 = jnp.maximum(j0, (e0 - base + (_L - 1)) // _L)

        def mbody(j, acc, lo=lo, hi=hi):
            pos = base + j * _L + lane
            v = vals_v[pl.ds(j * _L, _L)]
            m = (pos >= lo) & (pos < hi)
            return jnp.maximum(acc, jnp.where(m, v, _NEG_INF))

        acc = lax.fori_loop(j0, j1, mbody, jnp.full((_L,), _NEG_INF, jnp.float32))
        pm_v[pl.ds(b * _L, _L)] = acc

    pltpu.sync_copy(pm_v, shmax.at[pl.ds(sid * _PB, _PB)])

    pltpu.sync_copy(phi_hbm.at[pl.ds(wid * _CHW, _CHW)], minb_v)

    def nbody(j, acc):
        return jnp.minimum(acc, minb_v[pl.ds(j * _L, _L)])

    mn = lax.fori_loop(0, _CHW // _L, nbody, jnp.full((_L,), _POS_INF, jnp.float32))
    vec_v[...] = mn
    pltpu.sync_copy(vec_v, shmin.at[pl.ds(sid * _L, _L)])

    plsc.subcore_barrier()

    @pl.when(sid == 0)
    def _():
        pltpu.sync_copy(shmax, cmb_v)
        for b in range(_B):
            a = cmb_v[pl.ds(b * _L, _L)]
            for r in range(1, _NS):
                a = jnp.maximum(a, cmb_v[pl.ds(r * _PB + b * _L, _L)])
            pm_v[pl.ds(b * _L, _L)] = a
        pltpu.sync_copy(pm_v, outmax_hbm.at[pl.ds(cid * _PB, _PB)])

        pltpu.sync_copy(shmin, cmbmin_v)
        a = cmbmin_v[pl.ds(0, _L)]
        for r in range(1, _NS):
            a = jnp.minimum(a, cmbmin_v[pl.ds(r * _L, _L)])
        vec_v[...] = a
        pltpu.sync_copy(vec_v, outmin_hbm.at[pl.ds(cid * _L, _L)])


_sc_call = functools.partial(
    pl.kernel,
    out_type=[
        jax.ShapeDtypeStruct((_NC * _PB,), jnp.float32),
        jax.ShapeDtypeStruct((_NC * _L,), jnp.float32),
    ],
    scratch_types=[
        pltpu.VMEM((_L,), jnp.int32),
        pltpu.VMEM((_L,), jnp.int32),
        pltpu.VMEM((_TOK_W,), jnp.int32),
        pltpu.VMEM((_TOK_W,), jnp.float32),
        pltpu.VMEM((_CHW,), jnp.float32),
        pltpu.VMEM((_L,), jnp.float32),
        pltpu.VMEM((_PB,), jnp.float32),
        pltpu.VMEM_SHARED((_NS * _PB,), jnp.float32),
        pltpu.VMEM_SHARED((_NS * _L,), jnp.float32),
        pltpu.VMEM((_NS * _PB,), jnp.float32),
        pltpu.VMEM((_NS * _L,), jnp.float32),
        pltpu.SemaphoreType.DMA,
    ],
    mesh=plsc.VectorSubcoreMesh(core_axis_name="c", subcore_axis_name="s",
                                num_cores=_NC, num_subcores=_NS),
    name="aml_seg_max_sc",
)(_sc_body)


@jax.jit
def kernel(phi, flat_indices, cu_seqlens):
    phi = phi.astype(jnp.float32)
    idx = flat_indices.astype(jnp.int32)
    cu = cu_seqlens.astype(jnp.int32)
    phi_p = jnp.concatenate(
        [phi, jnp.full((_PHI_PAD - _NUM_ATOMS,), _POS_INF, jnp.float32)]
    )
    outmax, outmin = _sc_call(phi_p, idx, cu[:_B], cu[1:_B + 1])
    lengths = cu[1:] - cu[:-1]
    segmax = jnp.max(outmax.reshape(_NC, _B, _L), axis=(0, 2))
    traces = jnp.where(lengths == 0, jnp.min(outmin), segmax)
    return traces


# speedup vs baseline: 5.5315x; 1.0879x over previous
import functools
import jax
import jax.numpy as jnp
from jax import lax
from jax.experimental import pallas as pl
from jax.experimental.pallas import tpu as pltpu
from jax.experimental.pallas import tpu_sc as plsc

_NUM_ATOMS = 100000
_TOTAL = 32768
_B = 16
_NC = 2
_NS = 16
_L = 16
_NW = _NC * _NS
_TOK_W = _TOTAL // _NW
_GCH = 128
_NG = _TOK_W // _GCH
_CHW = 3136
_CHW_LAST = _NUM_ATOMS - _CHW
_PB = _B * _L

_NEG_INF = float("-inf")
_POS_INF = float("inf")


def _sc_body(phi_hbm, idx_hbm, culo_hbm, cuhi_hbm, outmax_hbm, outmin_hbm,
             culo_v, cuhi_v, idx_v, vals_v, minb_v, vec_v, pm_v, shmax,
             shmin, cmb_v, cmbmin_v, sem, sem_in, sem_min):
    cid = lax.axis_index("c")
    sid = lax.axis_index("s")
    wid = sid * _NC + cid
    base = wid * _TOK_W

    cp_idx = pltpu.async_copy(idx_hbm.at[pl.ds(base, _TOK_W)], idx_v, sem_in)
    moff = jnp.minimum(wid * _CHW, _CHW_LAST)
    cp_min = pltpu.async_copy(phi_hbm.at[pl.ds(moff, _CHW)], minb_v, sem_min)
    cp_lo = pltpu.async_copy(culo_hbm, culo_v, sem_in)
    cp_hi = pltpu.async_copy(cuhi_hbm, cuhi_v, sem_in)

    cp_idx.wait()
    cps = [
        pltpu.async_copy(
            phi_hbm.at[idx_v.at[pl.ds(j * _GCH, _GCH)]],
            vals_v.at[pl.ds(j * _GCH, _GCH)],
            sem,
        )
        for j in range(_NG)
    ]

    cp_min.wait()

    def nbody(j, acc):
        o = j * (4 * _L)
        acc = jnp.minimum(acc, minb_v[pl.ds(o, _L)])
        acc = jnp.minimum(acc, minb_v[pl.ds(o + _L, _L)])
        acc = jnp.minimum(acc, minb_v[pl.ds(o + 2 * _L, _L)])
        return jnp.minimum(acc, minb_v[pl.ds(o + 3 * _L, _L)])

    mn = lax.fori_loop(0, _CHW // (4 * _L), nbody,
                       jnp.full((_L,), _POS_INF, jnp.float32))
    vec_v[...] = mn
    pltpu.sync_copy(vec_v, shmin.at[pl.ds(sid * _L, _L)])

    cp_lo.wait()
    cp_hi.wait()
    for cp in cps:
        cp.wait()

    lane = lax.broadcasted_iota(jnp.int32, (_L,), 0)
    lov = culo_v[...]
    hiv = cuhi_v[...]

    for b in range(_B):
        lo = lov[b]
        hi = hiv[b]
        s0 = jnp.maximum(lo, base)
        e0 = jnp.minimum(hi, base + _TOK_W)
        j0 = (s0 - base) // _L
        j1 = jnp.maximum(j0, (e0 - base + (_L - 1)) // _L)

        def mbody(j, acc, lo=lo, hi=hi):
            pos = base + j * _L + lane
            v = vals_v[pl.ds(j * _L, _L)]
            m = (pos >= lo) & (pos < hi)
            return jnp.maximum(acc, jnp.where(m, v, _NEG_INF))

        acc = lax.fori_loop(j0, j1, mbody, jnp.full((_L,), _NEG_INF, jnp.float32))
        pm_v[pl.ds(b * _L, _L)] = acc

    pltpu.sync_copy(pm_v, shmax.at[pl.ds(sid * _PB, _PB)])

    plsc.subcore_barrier()

    @pl.when(sid == 0)
    def _():
        pltpu.sync_copy(shmax, cmb_v)
        for b in range(_B):
            a = cmb_v[pl.ds(b * _L, _L)]
            for r in range(1, _NS):
                a = jnp.maximum(a, cmb_v[pl.ds(r * _PB + b * _L, _L)])
            pm_v[pl.ds(b * _L, _L)] = a
        pltpu.sync_copy(pm_v, outmax_hbm.at[pl.ds(cid * _PB, _PB)])

        pltpu.sync_copy(shmin, cmbmin_v)
        a = cmbmin_v[pl.ds(0, _L)]
        for r in range(1, _NS):
            a = jnp.minimum(a, cmbmin_v[pl.ds(r * _L, _L)])
        vec_v[...] = a
        pltpu.sync_copy(vec_v, outmin_hbm.at[pl.ds(cid * _L, _L)])


_sc_call = functools.partial(
    pl.kernel,
    out_type=[
        jax.ShapeDtypeStruct((_NC * _PB,), jnp.float32),
        jax.ShapeDtypeStruct((_NC * _L,), jnp.float32),
    ],
    scratch_types=[
        pltpu.VMEM((_L,), jnp.int32),
        pltpu.VMEM((_L,), jnp.int32),
        pltpu.VMEM((_TOK_W,), jnp.int32),
        pltpu.VMEM((_TOK_W,), jnp.float32),
        pltpu.VMEM((_CHW,), jnp.float32),
        pltpu.VMEM((_L,), jnp.float32),
        pltpu.VMEM((_PB,), jnp.float32),
        pltpu.VMEM_SHARED((_NS * _PB,), jnp.float32),
        pltpu.VMEM_SHARED((_NS * _L,), jnp.float32),
        pltpu.VMEM((_NS * _PB,), jnp.float32),
        pltpu.VMEM((_NS * _L,), jnp.float32),
        pltpu.SemaphoreType.DMA,
        pltpu.SemaphoreType.DMA,
        pltpu.SemaphoreType.DMA,
    ],
    mesh=plsc.VectorSubcoreMesh(core_axis_name="c", subcore_axis_name="s",
                                num_cores=_NC, num_subcores=_NS),
    name="aml_seg_max_sc",
)(_sc_body)


@jax.jit
def kernel(phi, flat_indices, cu_seqlens):
    phi = phi.astype(jnp.float32)
    idx = flat_indices.astype(jnp.int32)
    cu = cu_seqlens.astype(jnp.int32)
    outmax, outmin = _sc_call(phi, idx, cu[:_B], cu[1:_B + 1])
    lengths = cu[1:] - cu[:-1]
    segmax = jnp.max(outmax.reshape(_NC, _B, _L), axis=(0, 2))
    traces = jnp.where(lengths == 0, jnp.min(outmin), segmax)
    return traces


# speedup vs baseline: 5.7383x; 1.0374x over previous
import functools
import jax
import jax.numpy as jnp
from jax import lax
from jax.experimental import pallas as pl
from jax.experimental.pallas import tpu as pltpu
from jax.experimental.pallas import tpu_sc as plsc

_NUM_ATOMS = 100000
_TOTAL = 32768
_B = 16
_NC = 2
_NS = 16
_L = 16
_NW = _NC * _NS
_TOK_W = _TOTAL // _NW
_GCH = 128
_NG = _TOK_W // _GCH
_CHW = 3136
_CHW_LAST = _NUM_ATOMS - _CHW
_PB = _B * _L

_NEG_INF = float("-inf")
_POS_INF = float("inf")


def _sc_body(phi_hbm, idx_hbm, cu_hbm, outmax_hbm, outmin_hbm,
             cu_v, idx_v, vals_v, minb_v, vec_v, pm_v, tseg_v,
             shmax, shmin, cmbmin_v, sem, sem_in, sem_min, sem_pub):
    cid = lax.axis_index("c")
    sid = lax.axis_index("s")
    wid = sid * _NC + cid
    base = wid * _TOK_W

    cp_idx = pltpu.async_copy(idx_hbm.at[pl.ds(base, _TOK_W)], idx_v, sem_in)
    moff = jnp.minimum(wid * _CHW, _CHW_LAST)
    cp_min = pltpu.async_copy(phi_hbm.at[pl.ds(moff, _CHW)], minb_v, sem_min)
    cp_cu = pltpu.async_copy(cu_hbm, cu_v, sem_in)

    cp_idx.wait()
    cp_g = pltpu.async_copy(phi_hbm.at[idx_v], vals_v, sem)

    cp_min.wait()

    def nbody(j, acc):
        o = j * (4 * _L)
        acc = jnp.minimum(acc, minb_v[pl.ds(o, _L)])
        acc = jnp.minimum(acc, minb_v[pl.ds(o + _L, _L)])
        acc = jnp.minimum(acc, minb_v[pl.ds(o + 2 * _L, _L)])
        return jnp.minimum(acc, minb_v[pl.ds(o + 3 * _L, _L)])

    mn = lax.fori_loop(0, _CHW // (4 * _L), nbody,
                       jnp.full((_L,), _POS_INF, jnp.float32))
    vec_v[...] = mn
    pltpu.sync_copy(vec_v, shmin.at[pl.ds(sid * _L, _L)])

    cp_cu.wait()
    lane = lax.broadcasted_iota(jnp.int32, (_L,), 0)
    lov = cu_v[pl.ds(0, _L)]
    hiv = cu_v[pl.ds(_L, _L)]
    cp_g.wait()

    pubs = []
    for b in range(_B):
        lo = lov[b]
        hi = hiv[b]
        s0 = jnp.maximum(lo, base)
        e0 = jnp.minimum(hi, base + _TOK_W)
        j0 = (s0 - base) // _L
        j1 = jnp.maximum(j0, (e0 - base + (_L - 1)) // _L)

        def mbody(j, acc, lo=lo, hi=hi):
            pos = base + j * _L + lane
            v = vals_v[pl.ds(j * _L, _L)]
            m = (pos >= lo) & (pos < hi)
            return jnp.maximum(acc, jnp.where(m, v, _NEG_INF))

        acc = lax.fori_loop(j0, j1, mbody, jnp.full((_L,), _NEG_INF, jnp.float32))
        pm_v[pl.ds(b * _L, _L)] = acc
        pubs.append(pltpu.async_copy(
            pm_v.at[pl.ds(b * _L, _L)],
            shmax.at[pl.ds(b * _NS * _L + sid * _L, _L)],
            sem_pub,
        ))
    for cp in pubs:
        cp.wait()

    plsc.subcore_barrier()

    pltpu.sync_copy(shmax.at[pl.ds(sid * _NS * _L, _NS * _L)], tseg_v)
    a = tseg_v[pl.ds(0, _L)]
    for r in range(1, _NS):
        a = jnp.maximum(a, tseg_v[pl.ds(r * _L, _L)])
    vec_v[...] = a
    pltpu.sync_copy(vec_v, outmax_hbm.at[pl.ds(cid * _PB + sid * _L, _L)])

    @pl.when(sid == 0)
    def _():
        pltpu.sync_copy(shmin, cmbmin_v)
        a = cmbmin_v[pl.ds(0, _L)]
        for r in range(1, _NS):
            a = jnp.minimum(a, cmbmin_v[pl.ds(r * _L, _L)])
        vec_v[...] = a
        pltpu.sync_copy(vec_v, outmin_hbm.at[pl.ds(cid * _L, _L)])


_sc_call = functools.partial(
    pl.kernel,
    out_type=[
        jax.ShapeDtypeStruct((_NC * _PB,), jnp.float32),
        jax.ShapeDtypeStruct((_NC * _L,), jnp.float32),
    ],
    scratch_types=[
        pltpu.VMEM((2 * _L,), jnp.int32),
        pltpu.VMEM((_TOK_W,), jnp.int32),
        pltpu.VMEM((_TOK_W,), jnp.float32),
        pltpu.VMEM((_CHW,), jnp.float32),
        pltpu.VMEM((_L,), jnp.float32),
        pltpu.VMEM((_PB,), jnp.float32),
        pltpu.VMEM((_NS * _L,), jnp.float32),
        pltpu.VMEM_SHARED((_B * _NS * _L,), jnp.float32),
        pltpu.VMEM_SHARED((_NS * _L,), jnp.float32),
        pltpu.VMEM((_NS * _L,), jnp.float32),
        pltpu.SemaphoreType.DMA,
        pltpu.SemaphoreType.DMA,
        pltpu.SemaphoreType.DMA,
        pltpu.SemaphoreType.DMA,
    ],
    mesh=plsc.VectorSubcoreMesh(core_axis_name="c", subcore_axis_name="s",
                                num_cores=_NC, num_subcores=_NS),
    name="aml_seg_max_sc",
)(_sc_body)


@jax.jit
def kernel(phi, flat_indices, cu_seqlens):
    phi = phi.astype(jnp.float32)
    idx = flat_indices.astype(jnp.int32)
    cu = cu_seqlens.astype(jnp.int32)
    cu2 = jnp.concatenate([cu[:_B], cu[1:_B + 1]])
    outmax, outmin = _sc_call(phi, idx, cu2)
    lengths = cu[1:] - cu[:-1]
    segmax = jnp.max(outmax.reshape(_NC, _B, _L), axis=(0, 2))
    traces = jnp.where(lengths == 0, jnp.min(outmin), segmax)
    return traces
